# baseline (speedup 1.0000x reference)
import functools

import jax
import jax.numpy as jnp
from jax import lax
from jax.experimental import pallas as pl
from jax.experimental.pallas import tpu as pltpu
from jax.experimental.pallas import tpu_sc as plsc

_NC = 2
_NS = 16
_NW = _NC * _NS
_CHUNK = 32
_NBUF = 3


@functools.lru_cache(maxsize=None)
def _make_gather(total_rows: int, d: int):
    rows_per_w = total_rows // _NW
    n_chunks = rows_per_w // _CHUNK
    mesh = plsc.VectorSubcoreMesh(core_axis_name="c", subcore_axis_name="s")

    @functools.partial(
        pl.kernel,
        mesh=mesh,
        out_type=jax.ShapeDtypeStruct((total_rows, d), jnp.float32),
        scratch_types=[
            pltpu.VMEM((rows_per_w,), jnp.int32),
            pltpu.VMEM((_NBUF, _CHUNK, d), jnp.float32),
            pltpu.SemaphoreType.DMA,
            pltpu.SemaphoreType.DMA,
        ],
    )
    def gather_kernel(idx_hbm, table_hbm, out_hbm, idx_v, bufs, gsem, ssem):
        wid = lax.axis_index("c") * _NS + lax.axis_index("s")
        base = wid * rows_per_w
        pltpu.sync_copy(idx_hbm.at[pl.ds(base, rows_per_w)], idx_v)

        def gather(c, b):
            pltpu.async_copy(
                table_hbm.at[idx_v.at[pl.ds(c * _CHUNK, _CHUNK)]],
                bufs.at[b], gsem,
            )

        def gather_wait(c, b):
            pltpu.make_async_copy(
                table_hbm.at[idx_v.at[pl.ds(c * _CHUNK, _CHUNK)]],
                bufs.at[b], gsem,
            ).wait()

        def scatter(c, b):
            pltpu.async_copy(
                bufs.at[b], out_hbm.at[pl.ds(base + c * _CHUNK, _CHUNK)],
                ssem,
            )

        def scatter_wait(c, b):
            pltpu.make_async_copy(
                bufs.at[b], out_hbm.at[pl.ds(base + c * _CHUNK, _CHUNK)],
                ssem,
            ).wait()

        gather(0, 0)

        def body(c, carry):
            cn = c + 1
            bn = lax.rem(cn, _NBUF)

            @pl.when(cn < n_chunks)
            def _():
                @pl.when(c >= _NBUF - 1)
                def _():
                    scatter_wait(cn - _NBUF, bn)

                gather(cn, bn)

            b = lax.rem(c, _NBUF)
            gather_wait(c, b)
            scatter(c, b)
            return carry

        lax.fori_loop(0, n_chunks, body, 0)
        for c in range(n_chunks - _NBUF, n_chunks):
            scatter_wait(c, c % _NBUF)

    return gather_kernel


def kernel(position_ids, pe):
    b, s = position_ids.shape
    idx = position_ids.reshape(-1).astype(jnp.int32)
    out = _make_gather(b * s, pe.shape[1])(idx, pe)
    return out.reshape(b, s, pe.shape[1])
